# Initial kernel scaffold; baseline (speedup 1.0000x reference)
import jax
import jax.numpy as jnp
from jax.experimental import pallas as pl


def kernel(des, tweet, num_prop, cat_prop, new_feature, edge_index, edge_type, W_des, b_des, W_tw, b_tw, W_np, b_np, W_cp, b_cp, W_nf, b_nf, W_in, b_in, Wr1, Wroot1, b1, Wr2, Wroot2, b2, W_o1, b_o1, W_o2, b_o2):
    raise NotImplementedError("write your pallas kernel here")


# speedup vs baseline: 6.1933x; 6.1933x over previous
import functools

import jax
import jax.numpy as jnp
from jax import lax
from jax.experimental import pallas as pl
from jax.experimental.pallas import tpu as pltpu
from jax.experimental.pallas import tpu_sc as plsc

_N = 10000
_E = 320000
_NT = 16
_NC = 2
_CH = 128
_CPT = 157
_EPAD = _NT * _CPT * _CH
_ER = _EPAD // _CH
_NTR = _N + 48
_AR = 2 * _NTR
_D2 = 64
_DC = 16
_STRIPE = _AR // _NT
_DIN = 1664
_BROW = 1000


def _leaky(x):
    return jnp.where(x > 0, x, 0.01 * x)


def _enc_body(inp_ref, we_ref, be_ref, wi_ref, bi_ref, out_ref):
    x = jnp.dot(inp_ref[...], we_ref[...], preferred_element_type=jnp.float32)
    x = _leaky(x + be_ref[...])
    x = jnp.dot(x, wi_ref[...], preferred_element_type=jnp.float32)
    out_ref[...] = _leaky(x + bi_ref[...])


def _idx_body(src_ref, dst_ref, typ_ref, g0_ref, g1_ref, w_ref):
    src = src_ref[...]
    g0_ref[...] = src * 2
    g1_ref[...] = src * 2 + 1
    t = typ_ref[...]
    lane = lax.broadcasted_iota(jnp.int32, t.shape, 1)
    w_ref[...] = jnp.where(t < 2, t * _NTR + dst_ref[...], _N + (lane & 7))


def _combine(xa_ref, p00, p01, p10, p11, c0, c1, wroot, b, wr0, wr1):
    y = jnp.dot(xa_ref[...], wroot[...], preferred_element_type=jnp.float32)
    y = y + b[...]
    for plo, phi, cr, wr in ((p00, p10, c0, wr0), (p01, p11, c1, wr1)):
        s = jnp.concatenate([plo[...], phi[...]], axis=1)
        s = s * (1.0 / jnp.maximum(cr[:, 0:1], 1.0))
        y = y + jnp.dot(s, wr[...], preferred_element_type=jnp.float32)
    return y


def _mid_body(xa_ref, p00, p01, p10, p11, c0, c1, wroot, b, wr0, wr1, out_ref):
    out_ref[...] = _combine(xa_ref, p00, p01, p10, p11, c0, c1,
                             wroot, b, wr0, wr1)


def _head_body(xa_ref, p00, p01, p10, p11, c0, c1, wroot, b, wr0, wr1,
               wo1, bo1, wo2, bo2, out_ref):
    y = _combine(xa_ref, p00, p01, p10, p11, c0, c1, wroot, b, wr0, wr1)
    h = _leaky(jnp.dot(y, wo1[...], preferred_element_type=jnp.float32) + bo1[...])
    out_ref[...] = jnp.dot(h, wo2[...], preferred_element_type=jnp.float32) + bo2[...]


def _row_spec(width):
    return pl.BlockSpec((_BROW, width), lambda i: (i, 0))


def _full_spec(shape):
    return pl.BlockSpec(shape, lambda i: tuple(0 for _ in shape))


def _sc_seg_body(x2, gidx, widx, zeros_hbm, out, gidx_v, widx_v, rows_v, acc, sem):
    c = lax.axis_index("c")
    s = lax.axis_index("s")
    r0 = s * _STRIPE
    pltpu.sync_copy(zeros_hbm, acc.at[pl.ds(r0, _STRIPE)])
    pltpu.sync_copy(gidx.at[c].at[s], gidx_v)
    pltpu.sync_copy(widx.at[s], widx_v)
    plsc.subcore_barrier()

    def step(j, carry):
        pltpu.async_copy(x2.at[gidx_v.at[j]], rows_v.at[0], sem).wait()
        pltpu.sync_copy(rows_v.at[0], acc.at[widx_v.at[j]], add=True)
        return carry

    lax.fori_loop(0, _CPT, step, 0)
    plsc.subcore_barrier()
    pltpu.sync_copy(acc.at[pl.ds(r0, _STRIPE)], out.at[c].at[pl.ds(r0, _STRIPE)])


def _sc_cnt_body(widx, ones_hbm, zeros_hbm, out, widx_v, ones_v, acc):
    c = lax.axis_index("c")
    s = lax.axis_index("s")
    r0 = s * _STRIPE
    pltpu.sync_copy(zeros_hbm, acc.at[pl.ds(r0, _STRIPE)])
    pltpu.sync_copy(ones_hbm, ones_v)
    pltpu.sync_copy(widx.at[s], widx_v)
    plsc.subcore_barrier()

    def step(j, carry):
        pltpu.sync_copy(ones_v, acc.at[widx_v.at[j]], add=True)
        return carry

    lax.fori_loop(0, _CPT, step, 0)
    plsc.subcore_barrier()
    pltpu.sync_copy(acc.at[pl.ds(r0, _STRIPE)], out.at[c].at[pl.ds(r0, _STRIPE)])


def _make_sc_fns():
    mesh = plsc.VectorSubcoreMesh(core_axis_name="c", subcore_axis_name="s")
    seg = pl.kernel(
        _sc_seg_body,
        out_type=jax.ShapeDtypeStruct((_NC, _AR, _D2), jnp.float32),
        mesh=mesh,
        scratch_types=[
            pltpu.VMEM((_CPT, _CH), jnp.int32),
            pltpu.VMEM((_CPT, _CH), jnp.int32),
            pltpu.VMEM((1, _CH, _D2), jnp.float32),
            pltpu.VMEM_SHARED((_AR, _D2), jnp.float32),
            pltpu.SemaphoreType.DMA,
        ],
        name="rgcn_seg_accumulate",
        compiler_params=pltpu.CompilerParams(use_tc_tiling_on_sc=False),
    )
    cnt = pl.kernel(
        _sc_cnt_body,
        out_type=jax.ShapeDtypeStruct((_NC, _AR, _DC), jnp.float32),
        mesh=mesh,
        scratch_types=[
            pltpu.VMEM((_CPT, _CH), jnp.int32),
            pltpu.VMEM((_CH, _DC), jnp.float32),
            pltpu.VMEM_SHARED((_AR, _DC), jnp.float32),
        ],
        name="rgcn_degree_count",
        compiler_params=pltpu.CompilerParams(use_tc_tiling_on_sc=False),
    )
    return seg, cnt


def _seg_accumulate(x2, gidx, widx, zeros_h):
    return _make_sc_fns()[0](x2, gidx, widx, zeros_h)


def _deg_count(widx, ones_h, zeros_h):
    return _make_sc_fns()[1](widx, ones_h, zeros_h)


def _split_acc(o):
    return (o[0, :_N], o[0, _NTR:_NTR + _N], o[1, :_N], o[1, _NTR:_NTR + _N])


def kernel(des, tweet, num_prop, cat_prop, new_feature, edge_index, edge_type,
           W_des, b_des, W_tw, b_tw, W_np, b_np, W_cp, b_cp, W_nf, b_nf,
           W_in, b_in, Wr1, Wroot1, b1, Wr2, Wroot2, b2, W_o1, b_o1, W_o2, b_o2):
    f32 = jnp.float32
    inp = jnp.concatenate([des, tweet, num_prop, cat_prop, new_feature], axis=1)
    inp = jnp.pad(inp, ((0, 0), (0, _DIN - 1555)))
    we = jnp.zeros((_DIN, 128), f32)
    we = we.at[0:768, 0:28].set(W_des)
    we = we.at[768:1536, 28:64].set(W_tw)
    we = we.at[1536:1543, 64:76].set(W_np)
    we = we.at[1543:1554, 76:116].set(W_cp)
    we = we.at[1554:1555, 116:128].set(W_nf)
    be = jnp.concatenate([b_des, b_tw, b_np, b_cp, b_nf]).reshape(1, 128)

    xenc = pl.pallas_call(
        _enc_body,
        grid=(_N // _BROW,),
        in_specs=[
            _row_spec(_DIN),
            _full_spec((_DIN, 128)),
            _full_spec((1, 128)),
            _full_spec((128, 128)),
            _full_spec((1, 128)),
        ],
        out_specs=_row_spec(128),
        out_shape=jax.ShapeDtypeStruct((_N, 128), f32),
    )(inp, we, be, W_in, b_in.reshape(1, 128))

    pad = _EPAD - _E
    src = jnp.pad(edge_index[0], (0, pad)).reshape(_ER, _CH)
    dst = jnp.pad(edge_index[1], (0, pad)).reshape(_ER, _CH)
    typ = jnp.pad(edge_type, (0, pad), constant_values=2).reshape(_ER, _CH)
    ispec = _full_spec((_ER, _CH))
    g0, g1, widx = pl.pallas_call(
        _idx_body,
        grid=(1,),
        in_specs=[ispec, ispec, ispec],
        out_specs=[ispec, ispec, ispec],
        out_shape=[jax.ShapeDtypeStruct((_ER, _CH), jnp.int32)] * 3,
    )(src, dst, typ)
    gidx = jnp.stack([g0, g1]).reshape(_NC, _NT, _CPT, _CH)
    widx = widx.reshape(_NT, _CPT, _CH)
    zeros_h = jnp.zeros((_STRIPE, _D2), f32)

    oc = _deg_count(widx, jnp.ones((_CH, _DC), f32), jnp.zeros((_STRIPE, _DC), f32))
    c0 = oc[0, :_N]
    c1 = oc[0, _NTR:_NTR + _N]

    o1 = _seg_accumulate(xenc.reshape(2 * _N, _D2), gidx, widx, zeros_h)
    wspec = _full_spec((128, 128))
    bspec = _full_spec((1, 128))
    pspec = _row_spec(_D2)
    cspec = _row_spec(_DC)
    x1 = pl.pallas_call(
        _mid_body,
        grid=(_N // _BROW,),
        in_specs=[_row_spec(128), pspec, pspec, pspec, pspec, cspec, cspec,
                  wspec, bspec, wspec, wspec],
        out_specs=_row_spec(128),
        out_shape=jax.ShapeDtypeStruct((_N, 128), f32),
    )(xenc, *_split_acc(o1), c0, c1, Wroot1, b1.reshape(1, 128), Wr1[0], Wr1[1])

    o2 = _seg_accumulate(x1.reshape(2 * _N, _D2), gidx, widx, zeros_h)
    wo2 = jnp.zeros((128, 128), f32).at[:, :2].set(W_o2)
    bo2 = jnp.zeros((1, 128), f32).at[0, :2].set(b_o2)
    out = pl.pallas_call(
        _head_body,
        grid=(_N // _BROW,),
        in_specs=[_row_spec(128), pspec, pspec, pspec, pspec, cspec, cspec,
                  wspec, bspec, wspec, wspec,
                  wspec, bspec, wspec, bspec],
        out_specs=_row_spec(128),
        out_shape=jax.ShapeDtypeStruct((_N, 128), f32),
    )(x1, *_split_acc(o2), c0, c1, Wroot2, b2.reshape(1, 128), Wr2[0], Wr2[1],
      W_o1, b_o1.reshape(1, 128), wo2, bo2)
    return out[:, :2]
